# baseline (speedup 1.0000x reference)
import functools

import jax
import jax.numpy as jnp
import numpy as np
from jax import lax
from jax.experimental import pallas as pl
from jax.experimental.pallas import tpu as pltpu
from jax.experimental.pallas import tpu_sc as plsc

MAX_LEN = 200
EMB = 64
NC = 2
NS = 16
NW = NC * NS
UB = 256
NBB = 4096 // 128


def _positional_signal_np(hidden_size: int, length: int) -> np.ndarray:
    position = np.arange(0, length, dtype=np.float32)
    num_timescales = hidden_size // 2
    log_inc = np.float32(np.log(10000.0) / (num_timescales - 1))
    inv_timescales = np.exp(np.arange(num_timescales, dtype=np.float32) * -log_inc)
    scaled = position[:, None] * inv_timescales[None, :]
    return np.concatenate([np.sin(scaled), np.cos(scaled)], axis=1).astype(np.float32)


def _make_kernel(batch: int):
    units_per_tile = (MAX_LEN * batch // UB) // NW
    mesh = plsc.VectorSubcoreMesh(core_axis_name="c", subcore_axis_name="s")

    @functools.partial(
        pl.kernel,
        mesh=mesh,
        compiler_params=pltpu.CompilerParams(
            use_tc_tiling_on_sc=False, needs_layout_passes=False
        ),
        out_type=jax.ShapeDtypeStruct((MAX_LEN, EMB // 8, NBB, 8, 128), jnp.float32),
        scratch_types=[
            pltpu.VMEM((64, 16), jnp.float32),
        ]
        + [pltpu.VMEM((2, 128), jnp.int32)] * 2
        + [pltpu.VMEM((UB, EMB), jnp.float32)] * 2
        + [pltpu.VMEM((EMB // 8, 2, 8, 128), jnp.float32)] * 2
        + [pltpu.SemaphoreType.DMA] * 4,
    )
    def k(ids_hbm, sigb_hbm, table_hbm, out_hbm, sigb_v, *bufs):
        idxs = bufs[0:2]
        raws = bufs[2:4]
        obufs = bufs[4:6]
        gsem = bufs[6:8]
        ssem = bufs[8:10]
        tid = lax.axis_index("s") * NC + lax.axis_index("c")
        q = lax.rem(tid, 16)
        l0 = tid // 16

        iota = lax.iota(jnp.int32, 16)

        def unit_l(k_):
            return l0 + 2 * k_

        def stage_and_gather(k_, b):
            l = unit_l(k_)
            pltpu.sync_copy(ids_hbm.at[l, pl.ds(2 * q, 2)], idxs[b])
            for j in range(2):
                pltpu.async_copy(
                    table_hbm.at[idxs[b].at[j]],
                    raws[b].at[pl.ds(j * 128, 128)],
                    gsem[b],
                )

        def wait_gather(k_, b):
            for j in range(2):
                pltpu.make_async_copy(
                    table_hbm.at[idxs[b].at[j]],
                    raws[b].at[pl.ds(j * 128, 128)],
                    gsem[b],
                ).wait()

        def start_stores(k_, b):
            l = unit_l(k_)
            for e8 in range(EMB // 8):
                pltpu.async_copy(
                    obufs[b].at[e8], out_hbm.at[l, e8, pl.ds(2 * q, 2)], ssem[b]
                )

        def wait_stores(k_, b):
            l = unit_l(k_)
            for e8 in range(EMB // 8):
                pltpu.make_async_copy(
                    obufs[b].at[e8], out_hbm.at[l, e8, pl.ds(2 * q, 2)], ssem[b]
                ).wait()

        def compute(k_, b):
            l = unit_l(k_)
            pltpu.sync_copy(sigb_hbm.at[l], sigb_v)
            raw = raws[b]
            obuf = obufs[b]
            for e in range(EMB):
                sv = sigb_v[e, :]
                colv = jnp.full((16,), e, jnp.int32)
                e8, e1 = e // 8, e % 8

                @plsc.parallel_loop(0, UB // 16, 1, unroll=4)
                def _g(g):
                    vals = plsc.load_gather(raw, [iota + 16 * g, colv])
                    obuf[e8, g // 8, e1, pl.ds(lax.rem(g, 8) * 16, 16)] = vals + sv

        stage_and_gather(0, 0)

        def macro(m, carry):
            for b in range(2):
                k_ = m * 2 + b
                pl.when(k_ + 1 < units_per_tile)(
                    lambda: stage_and_gather(k_ + 1, (b + 1) % 2)
                )
                wait_gather(k_, b)
                pl.when(k_ >= 2)(lambda: wait_stores(k_ - 2, b))
                compute(k_, b)
                start_stores(k_, b)
            return carry

        lax.fori_loop(0, units_per_tile // 2, macro, 0, unroll=False)
        for b in range(2):
            wait_stores(units_per_tile - 2 + b, b)

    return k


def kernel(x, table):
    batch, length = x.shape
    ids = x.T.reshape(MAX_LEN, batch // 128, 128).astype(jnp.int32)
    sig = _positional_signal_np(EMB, MAX_LEN)
    sigb = jnp.asarray(np.repeat(sig[:, :, None], 16, axis=2))
    out5 = _make_kernel(batch)(ids, sigb, table)
    return out5.transpose(2, 4, 0, 1, 3).reshape(batch, MAX_LEN, EMB)


# speedup vs baseline: 1.4427x; 1.4427x over previous
import functools

import jax
import jax.numpy as jnp
import numpy as np
from jax import lax
from jax.experimental import pallas as pl
from jax.experimental.pallas import tpu as pltpu
from jax.experimental.pallas import tpu_sc as plsc

MAX_LEN = 200
EMB = 64
NC = 2
NS = 16
NW = NC * NS
UB = 256
NBB = 4096 // 128


def _positional_signal_np(hidden_size: int, length: int) -> np.ndarray:
    position = np.arange(0, length, dtype=np.float32)
    num_timescales = hidden_size // 2
    log_inc = np.float32(np.log(10000.0) / (num_timescales - 1))
    inv_timescales = np.exp(np.arange(num_timescales, dtype=np.float32) * -log_inc)
    scaled = position[:, None] * inv_timescales[None, :]
    return np.concatenate([np.sin(scaled), np.cos(scaled)], axis=1).astype(np.float32)


def _make_kernel(batch: int):
    units_per_tile = (MAX_LEN * batch // UB) // NW
    mesh = plsc.VectorSubcoreMesh(core_axis_name="c", subcore_axis_name="s")

    @functools.partial(
        pl.kernel,
        mesh=mesh,
        compiler_params=pltpu.CompilerParams(
            use_tc_tiling_on_sc=False, needs_layout_passes=False
        ),
        out_type=jax.ShapeDtypeStruct((MAX_LEN, EMB // 8, NBB, 8, 128), jnp.float32),
        scratch_types=[
            pltpu.VMEM((64, 16), jnp.float32),
        ]
        + [pltpu.VMEM((2, 128), jnp.int32)] * 2
        + [pltpu.VMEM((UB, EMB), jnp.float32)] * 2
        + [pltpu.VMEM((UB * (EMB + 1),), jnp.float32)]
        + [pltpu.VMEM((EMB // 8, 2, 8, 128), jnp.float32)] * 2
        + [pltpu.SemaphoreType.DMA] * 4,
    )
    def k(ids_hbm, sigb_hbm, table_hbm, out_hbm, sigb_v, *bufs):
        idxs = bufs[0:2]
        raws = bufs[2:4]
        rawp = bufs[4]
        obufs = bufs[5:7]
        gsem = bufs[7:9]
        ssem = bufs[9:11]
        tid = lax.axis_index("s") * NC + lax.axis_index("c")
        q = lax.rem(tid, 16)
        l0 = tid // 16

        iota = lax.iota(jnp.int32, 16)

        def unit_l(k_):
            return l0 + 2 * k_

        def stage_and_gather(k_, b):
            l = unit_l(k_)
            pltpu.sync_copy(ids_hbm.at[l, pl.ds(2 * q, 2)], idxs[b])
            for j in range(2):
                pltpu.async_copy(
                    table_hbm.at[idxs[b].at[j]],
                    raws[b].at[pl.ds(j * 128, 128)],
                    gsem[b],
                )

        def wait_gather(k_, b):
            for j in range(2):
                pltpu.make_async_copy(
                    table_hbm.at[idxs[b].at[j]],
                    raws[b].at[pl.ds(j * 128, 128)],
                    gsem[b],
                ).wait()

        def start_stores(k_, b):
            l = unit_l(k_)
            for e8 in range(EMB // 8):
                pltpu.async_copy(
                    obufs[b].at[e8], out_hbm.at[l, e8, pl.ds(2 * q, 2)], ssem[b]
                )

        def wait_stores(k_, b):
            l = unit_l(k_)
            for e8 in range(EMB // 8):
                pltpu.make_async_copy(
                    obufs[b].at[e8], out_hbm.at[l, e8, pl.ds(2 * q, 2)], ssem[b]
                ).wait()

        PITCH = EMB + 1
        iot65 = iota * PITCH

        def compute(k_, b):
            l = unit_l(k_)
            pltpu.sync_copy(sigb_hbm.at[l], sigb_v)
            raw = raws[b]
            obuf = obufs[b]

            @plsc.parallel_loop(0, UB, 1, unroll=2)
            def _r(r):
                o = r * PITCH
                for j in range(EMB // 16):
                    rawp[pl.ds(o + 16 * j, 16)] = raw[r, pl.ds(16 * j, 16)]

            for e in range(EMB):
                sv = sigb_v[e, :]
                e8, e1 = e // 8, e % 8

                @plsc.parallel_loop(0, UB // 16, 1, unroll=4)
                def _g(g):
                    s = 16 * PITCH * g + e
                    vals = plsc.load_gather(rawp, [iot65 + s])
                    obuf[e8, g // 8, e1, pl.ds(lax.rem(g, 8) * 16, 16)] = vals + sv

        stage_and_gather(0, 0)

        def macro(m, carry):
            for b in range(2):
                k_ = m * 2 + b
                pl.when(k_ + 1 < units_per_tile)(
                    lambda: stage_and_gather(k_ + 1, (b + 1) % 2)
                )
                wait_gather(k_, b)
                pl.when(k_ >= 2)(lambda: wait_stores(k_ - 2, b))
                compute(k_, b)
                start_stores(k_, b)
            return carry

        lax.fori_loop(0, units_per_tile // 2, macro, 0, unroll=False)
        for b in range(2):
            wait_stores(units_per_tile - 2 + b, b)

    return k


def kernel(x, table):
    batch, length = x.shape
    ids = x.T.reshape(MAX_LEN, batch // 128, 128).astype(jnp.int32)
    sig = _positional_signal_np(EMB, MAX_LEN)
    sigb = jnp.asarray(np.repeat(sig[:, :, None], 16, axis=2))
    out5 = _make_kernel(batch)(ids, sigb, table)
    return out5.transpose(2, 4, 0, 1, 3).reshape(batch, MAX_LEN, EMB)


# speedup vs baseline: 1.7414x; 1.2071x over previous
import functools

import jax
import jax.numpy as jnp
import numpy as np
from jax import lax
from jax.experimental import pallas as pl
from jax.experimental.pallas import tpu as pltpu
from jax.experimental.pallas import tpu_sc as plsc

MAX_LEN = 200
EMB = 64
NC = 2
NS = 16
NW = NC * NS
UB = 256
NBB = 4096 // 128
PITCH = EMB + 1


def _positional_signal_np(hidden_size: int, length: int) -> np.ndarray:
    position = np.arange(0, length, dtype=np.float32)
    num_timescales = hidden_size // 2
    log_inc = np.float32(np.log(10000.0) / (num_timescales - 1))
    inv_timescales = np.exp(np.arange(num_timescales, dtype=np.float32) * -log_inc)
    scaled = position[:, None] * inv_timescales[None, :]
    return np.concatenate([np.sin(scaled), np.cos(scaled)], axis=1).astype(np.float32)


def _lane_broadcast(vec, lane):
    idxs = jnp.full((16, 1), lane, jnp.int32)
    dn = lax.GatherDimensionNumbers(
        offset_dims=(), collapsed_slice_dims=(0,), start_index_map=(0,)
    )
    return lax.gather(
        vec, idxs, dn, (1,), mode=lax.GatherScatterMode.PROMISE_IN_BOUNDS
    )


def _make_kernel(batch: int):
    units_per_tile = (MAX_LEN * batch // UB) // NW
    mesh = plsc.VectorSubcoreMesh(core_axis_name="c", subcore_axis_name="s")

    @functools.partial(
        pl.kernel,
        mesh=mesh,
        compiler_params=pltpu.CompilerParams(
            use_tc_tiling_on_sc=False, needs_layout_passes=False
        ),
        out_type=jax.ShapeDtypeStruct((MAX_LEN, EMB // 8, NBB, 8, 128), jnp.float32),
        scratch_types=[
            pltpu.VMEM((MAX_LEN // 2, 2, 128), jnp.int32),
            pltpu.VMEM((MAX_LEN * EMB,), jnp.float32),
        ]
        + [pltpu.VMEM((UB, EMB), jnp.float32)] * 2
        + [pltpu.VMEM((UB * PITCH,), jnp.float32)]
        + [pltpu.VMEM((EMB // 8, 2, 8, 128), jnp.float32)] * 2
        + [pltpu.SemaphoreType.DMA] * 4,
    )
    def k(ids_hbm, sig_hbm, table_hbm, out_hbm, idall_v, sig_v, *bufs):
        raws = bufs[0:2]
        rawp = bufs[2]
        obufs = bufs[3:5]
        gsem = bufs[5:7]
        ssem = bufs[7:9]
        tid = lax.axis_index("s") * NC + lax.axis_index("c")
        q = lax.rem(tid, 16)
        l0 = tid // 16

        iota = lax.iota(jnp.int32, 16)
        iotp = iota * PITCH

        pltpu.sync_copy(ids_hbm.at[l0, q], idall_v)
        pltpu.sync_copy(sig_hbm, sig_v)

        def unit_l(k_):
            return l0 + 2 * k_

        def start_gather(k_, b):
            for j in range(2):
                pltpu.async_copy(
                    table_hbm.at[idall_v.at[k_, j]],
                    raws[b].at[pl.ds(j * 128, 128)],
                    gsem[b],
                )

        def wait_gather(k_, b):
            for j in range(2):
                pltpu.make_async_copy(
                    table_hbm.at[idall_v.at[k_, j]],
                    raws[b].at[pl.ds(j * 128, 128)],
                    gsem[b],
                ).wait()

        def start_stores(k_, b):
            l = unit_l(k_)
            for e8 in range(EMB // 8):
                pltpu.async_copy(
                    obufs[b].at[e8], out_hbm.at[l, e8, pl.ds(2 * q, 2)], ssem[b]
                )

        def wait_stores(k_, b):
            l = unit_l(k_)
            for e8 in range(EMB // 8):
                pltpu.make_async_copy(
                    obufs[b].at[e8], out_hbm.at[l, e8, pl.ds(2 * q, 2)], ssem[b]
                ).wait()

        def compute(k_, b):
            l = unit_l(k_)
            raw = raws[b]
            obuf = obufs[b]

            @plsc.parallel_loop(0, UB, 1, unroll=2)
            def _r(r):
                o = r * PITCH
                for j in range(EMB // 16):
                    rawp[pl.ds(o + 16 * j, 16)] = raw[r, pl.ds(16 * j, 16)]

            svecs = [sig_v[pl.ds(l * EMB + 16 * t, 16)] for t in range(EMB // 16)]
            for e in range(EMB):
                sv = _lane_broadcast(svecs[e // 16], e % 16)
                e8, e1 = e // 8, e % 8

                @plsc.parallel_loop(0, UB // 16, 1, unroll=4)
                def _g(g):
                    s = 16 * PITCH * g + e
                    vals = plsc.load_gather(rawp, [iotp + s])
                    obuf[e8, g // 8, e1, pl.ds(lax.rem(g, 8) * 16, 16)] = vals + sv

        start_gather(0, 0)

        def macro(m, carry):
            for b in range(2):
                k_ = m * 2 + b
                pl.when(k_ + 1 < units_per_tile)(
                    lambda: start_gather(k_ + 1, (b + 1) % 2)
                )
                wait_gather(k_, b)
                pl.when(k_ >= 2)(lambda: wait_stores(k_ - 2, b))
                compute(k_, b)
                start_stores(k_, b)
            return carry

        lax.fori_loop(0, units_per_tile // 2, macro, 0, unroll=False)
        for b in range(2):
            wait_stores(units_per_tile - 2 + b, b)

    return k


def kernel(x, table):
    batch, length = x.shape
    ids = (
        x.T.astype(jnp.int32)
        .reshape(MAX_LEN // 2, 2, 16, 2, 128)
        .transpose(1, 2, 0, 3, 4)
    )
    sig = jnp.asarray(_positional_signal_np(EMB, MAX_LEN).reshape(-1))
    out5 = _make_kernel(batch)(ids, sig, table)
    return out5.transpose(2, 4, 0, 1, 3).reshape(batch, MAX_LEN, EMB)


# speedup vs baseline: 1.8354x; 1.0540x over previous
import functools

import jax
import jax.numpy as jnp
import numpy as np
from jax import lax
from jax.experimental import pallas as pl
from jax.experimental.pallas import tpu as pltpu
from jax.experimental.pallas import tpu_sc as plsc

MAX_LEN = 200
EMB = 64
NC = 2
NS = 16
NW = NC * NS
UB = 256
NBB = 4096 // 128
PITCH = EMB + 1


def _positional_signal_np(hidden_size: int, length: int) -> np.ndarray:
    position = np.arange(0, length, dtype=np.float32)
    num_timescales = hidden_size // 2
    log_inc = np.float32(np.log(10000.0) / (num_timescales - 1))
    inv_timescales = np.exp(np.arange(num_timescales, dtype=np.float32) * -log_inc)
    scaled = position[:, None] * inv_timescales[None, :]
    return np.concatenate([np.sin(scaled), np.cos(scaled)], axis=1).astype(np.float32)


def _lane_broadcast(vec, lane):
    idxs = jnp.full((16, 1), lane, jnp.int32)
    dn = lax.GatherDimensionNumbers(
        offset_dims=(), collapsed_slice_dims=(0,), start_index_map=(0,)
    )
    return lax.gather(
        vec, idxs, dn, (1,), mode=lax.GatherScatterMode.PROMISE_IN_BOUNDS
    )


def _make_kernel(batch: int):
    units_per_tile = (MAX_LEN * batch // UB) // NW
    mesh = plsc.VectorSubcoreMesh(core_axis_name="c", subcore_axis_name="s")

    @functools.partial(
        pl.kernel,
        mesh=mesh,
        compiler_params=pltpu.CompilerParams(
            use_tc_tiling_on_sc=False, needs_layout_passes=False
        ),
        out_type=jax.ShapeDtypeStruct((MAX_LEN, EMB // 8, NBB, 8, 128), jnp.float32),
        scratch_types=[
            pltpu.VMEM((MAX_LEN // 2, 2, 128), jnp.int32),
            pltpu.VMEM((MAX_LEN * EMB,), jnp.float32),
        ]
        + [pltpu.VMEM((UB, EMB), jnp.float32)] * 2
        + [pltpu.VMEM((UB * PITCH,), jnp.float32)]
        + [pltpu.VMEM((EMB // 8, 2, 8, 128), jnp.float32)] * 2
        + [pltpu.SemaphoreType.DMA] * 4,
    )
    def k(ids_hbm, sig_hbm, table_hbm, out_hbm, idall_v, sig_v, *bufs):
        raws = bufs[0:2]
        rawp = bufs[2]
        obufs = bufs[3:5]
        gsem = bufs[5:7]
        ssem = bufs[7:9]
        tid = lax.axis_index("s") * NC + lax.axis_index("c")
        q = lax.rem(tid, 16)
        l0 = tid // 16

        iota = lax.iota(jnp.int32, 16)
        iotp = iota * PITCH

        pltpu.sync_copy(ids_hbm.at[l0, q], idall_v)
        pltpu.sync_copy(sig_hbm, sig_v)

        def unit_l(k_):
            return l0 + 2 * k_

        def start_gather(k_, b):
            for j in range(2):
                pltpu.async_copy(
                    table_hbm.at[idall_v.at[k_, j]],
                    raws[b].at[pl.ds(j * 128, 128)],
                    gsem[b],
                )

        def wait_gather(k_, b):
            for j in range(2):
                pltpu.make_async_copy(
                    table_hbm.at[idall_v.at[k_, j]],
                    raws[b].at[pl.ds(j * 128, 128)],
                    gsem[b],
                ).wait()

        def start_stores(k_, b):
            l = unit_l(k_)
            for e8 in range(EMB // 8):
                pltpu.async_copy(
                    obufs[b].at[e8], out_hbm.at[l, e8, pl.ds(2 * q, 2)], ssem[b]
                )

        def wait_stores(k_, b):
            l = unit_l(k_)
            for e8 in range(EMB // 8):
                pltpu.make_async_copy(
                    obufs[b].at[e8], out_hbm.at[l, e8, pl.ds(2 * q, 2)], ssem[b]
                ).wait()

        def compute(k_, b):
            l = unit_l(k_)
            raw = raws[b]
            obuf = obufs[b]

            @plsc.parallel_loop(0, UB, 1, unroll=4)
            def _r(r):
                o = r * PITCH
                for j in range(EMB // 16):
                    rawp[pl.ds(o + 16 * j, 16)] = raw[r, pl.ds(16 * j, 16)]

            svecs = [sig_v[pl.ds(l * EMB + 16 * t, 16)] for t in range(EMB // 16)]
            for e in range(EMB):
                sv = _lane_broadcast(svecs[e // 16], e % 16)
                e8, e1 = e // 8, e % 8

                @plsc.parallel_loop(0, UB // 16, 1, unroll=8)
                def _g(g):
                    s = 16 * PITCH * g + e
                    vals = plsc.load_gather(rawp, [iotp + s])
                    obuf[e8, g // 8, e1, pl.ds(lax.rem(g, 8) * 16, 16)] = vals + sv

        start_gather(0, 0)

        def macro(m, carry):
            for b in range(2):
                k_ = m * 2 + b
                pl.when(k_ + 1 < units_per_tile)(
                    lambda: start_gather(k_ + 1, (b + 1) % 2)
                )
                wait_gather(k_, b)
                pl.when(k_ >= 2)(lambda: wait_stores(k_ - 2, b))
                compute(k_, b)
                start_stores(k_, b)
            return carry

        lax.fori_loop(0, units_per_tile // 2, macro, 0, unroll=False)
        for b in range(2):
            wait_stores(units_per_tile - 2 + b, b)

    return k


def kernel(x, table):
    batch, length = x.shape
    ids = (
        x.T.astype(jnp.int32)
        .reshape(MAX_LEN // 2, 2, 16, 2, 128)
        .transpose(1, 2, 0, 3, 4)
    )
    sig = jnp.asarray(_positional_signal_np(EMB, MAX_LEN).reshape(-1))
    out5 = _make_kernel(batch)(ids, sig, table)
    return out5.transpose(2, 4, 0, 1, 3).reshape(batch, MAX_LEN, EMB)


# speedup vs baseline: 1.8382x; 1.0015x over previous
import functools

import jax
import jax.numpy as jnp
import numpy as np
from jax import lax
from jax.experimental import pallas as pl
from jax.experimental.pallas import tpu as pltpu
from jax.experimental.pallas import tpu_sc as plsc

MAX_LEN = 200
EMB = 64
NC = 2
NS = 16
NW = NC * NS
UB = 256
NBB = 4096 // 128
PITCH = EMB + 1


def _positional_signal_np(hidden_size: int, length: int) -> np.ndarray:
    position = np.arange(0, length, dtype=np.float32)
    num_timescales = hidden_size // 2
    log_inc = np.float32(np.log(10000.0) / (num_timescales - 1))
    inv_timescales = np.exp(np.arange(num_timescales, dtype=np.float32) * -log_inc)
    scaled = position[:, None] * inv_timescales[None, :]
    return np.concatenate([np.sin(scaled), np.cos(scaled)], axis=1).astype(np.float32)


def _lane_broadcast(vec, lane):
    idxs = jnp.full((16, 1), lane, jnp.int32)
    dn = lax.GatherDimensionNumbers(
        offset_dims=(), collapsed_slice_dims=(0,), start_index_map=(0,)
    )
    return lax.gather(
        vec, idxs, dn, (1,), mode=lax.GatherScatterMode.PROMISE_IN_BOUNDS
    )


def _make_kernel(batch: int):
    units_per_tile = (MAX_LEN * batch // UB) // NW
    mesh = plsc.VectorSubcoreMesh(core_axis_name="c", subcore_axis_name="s")

    @functools.partial(
        pl.kernel,
        mesh=mesh,
        compiler_params=pltpu.CompilerParams(
            use_tc_tiling_on_sc=False, needs_layout_passes=False
        ),
        out_type=jax.ShapeDtypeStruct((MAX_LEN, EMB // 8, NBB, 8, 128), jnp.float32),
        scratch_types=[
            pltpu.VMEM((MAX_LEN // 2, 2, 128), jnp.int32),
            pltpu.VMEM((MAX_LEN * EMB,), jnp.float32),
        ]
        + [pltpu.VMEM((UB, EMB), jnp.float32)] * 2
        + [pltpu.VMEM((UB * PITCH,), jnp.float32)]
        + [pltpu.VMEM((EMB // 8, 2, 8, 128), jnp.float32)] * 2
        + [pltpu.SemaphoreType.DMA] * 4,
    )
    def k(ids_hbm, sig_hbm, table_hbm, out_hbm, idall_v, sig_v, *bufs):
        raws = bufs[0:2]
        rawp = bufs[2]
        obufs = bufs[3:5]
        gsem = bufs[5:7]
        ssem = bufs[7:9]
        tid = lax.axis_index("s") * NC + lax.axis_index("c")
        q = lax.rem(tid, 16)
        l0 = tid // 16

        iota = lax.iota(jnp.int32, 16)
        iotp = iota * PITCH

        pltpu.sync_copy(ids_hbm.at[l0, q], idall_v)
        pltpu.sync_copy(sig_hbm, sig_v)

        def unit_l(k_):
            return l0 + 2 * k_

        def start_gather(k_, b):
            for j in range(2):
                pltpu.async_copy(
                    table_hbm.at[idall_v.at[k_, j]],
                    raws[b].at[pl.ds(j * 128, 128)],
                    gsem[b],
                )

        def wait_gather(k_, b):
            for j in range(2):
                pltpu.make_async_copy(
                    table_hbm.at[idall_v.at[k_, j]],
                    raws[b].at[pl.ds(j * 128, 128)],
                    gsem[b],
                ).wait()

        def start_stores(k_, b):
            l = unit_l(k_)
            for e8 in range(EMB // 8):
                pltpu.async_copy(
                    obufs[b].at[e8], out_hbm.at[l, e8, pl.ds(2 * q, 2)], ssem[b]
                )

        def wait_stores(k_, b):
            l = unit_l(k_)
            for e8 in range(EMB // 8):
                pltpu.make_async_copy(
                    obufs[b].at[e8], out_hbm.at[l, e8, pl.ds(2 * q, 2)], ssem[b]
                ).wait()

        def compute(k_, b):
            l = unit_l(k_)
            raw = raws[b]
            obuf = obufs[b]

            @plsc.parallel_loop(0, UB, 1, unroll=8)
            def _r(r):
                o = r * PITCH
                for j in range(EMB // 16):
                    rawp[pl.ds(o + 16 * j, 16)] = raw[r, pl.ds(16 * j, 16)]

            svecs = [sig_v[pl.ds(l * EMB + 16 * t, 16)] for t in range(EMB // 16)]
            for e in range(EMB):
                sv = _lane_broadcast(svecs[e // 16], e % 16)
                e8, e1 = e // 8, e % 8

                @plsc.parallel_loop(0, UB // 16, 1, unroll=8)
                def _g(g):
                    s = 16 * PITCH * g + e
                    vals = plsc.load_gather(rawp, [iotp + s])
                    obuf[e8, g // 8, e1, pl.ds(lax.rem(g, 8) * 16, 16)] = vals + sv

        start_gather(0, 0)

        def macro(m, carry):
            for b in range(2):
                k_ = m * 2 + b
                pl.when(k_ + 1 < units_per_tile)(
                    lambda: start_gather(k_ + 1, (b + 1) % 2)
                )
                wait_gather(k_, b)
                pl.when(k_ >= 2)(lambda: wait_stores(k_ - 2, b))
                compute(k_, b)
                start_stores(k_, b)
            return carry

        lax.fori_loop(0, units_per_tile // 2, macro, 0, unroll=False)
        for b in range(2):
            wait_stores(units_per_tile - 2 + b, b)

    return k


def kernel(x, table):
    batch, length = x.shape
    ids = (
        x.T.astype(jnp.int32)
        .reshape(MAX_LEN // 2, 2, 16, 2, 128)
        .transpose(1, 2, 0, 3, 4)
    )
    sig = jnp.asarray(_positional_signal_np(EMB, MAX_LEN).reshape(-1))
    out5 = _make_kernel(batch)(ids, sig, table)
    return out5.transpose(2, 4, 0, 1, 3).reshape(batch, MAX_LEN, EMB)
